# baseline (speedup 1.0000x reference)
import functools

import jax
import jax.numpy as jnp
from jax import lax
from jax.experimental import pallas as pl
from jax.experimental.pallas import tpu as pltpu
from jax.experimental.pallas import tpu_sc as plsc

_VOCAB = 50257
_D_MODEL = 768
_B = 4
_S = 8192
_N = _B * _S

_NC = 2
_NS = 16
_NW = _NC * _NS
_BPW = _N // _NW
_CH = 128
_NCHUNK = _BPW // _CH


def _sc_gather_body(ids_hbm, table_hbm, out_hbm, idx_v, offs_v, g_v, sem):
    wid = lax.axis_index("s") * _NC + lax.axis_index("c")
    base = wid * _BPW
    bidx = wid // 8
    ccol0 = 8 * (wid % 8)
    pltpu.sync_copy(ids_hbm.at[pl.ds(ccol0, 8), pl.ds(bidx, 1), :], idx_v)
    for j in range(_NCHUNK):
        for k in range(8):
            sl = pl.ds(16 * k, 16)
            t = idx_v[j, 0, sl]
            for ch in range(3):
                offs_v[ch, j, sl] = t + ch * _VOCAB
    copies = []
    for j in range(_NCHUNK):
        for ch in range(3):
            copies.append(
                pltpu.async_copy(
                    table_hbm.at[offs_v.at[ch, j]],
                    g_v.at[ch, pl.ds(_CH * j, _CH)],
                    sem,
                )
            )
    for c in copies:
        c.wait()
    for ch in range(3):
        pltpu.sync_copy(g_v.at[ch], out_hbm.at[pl.ds(ch * _N + base, _BPW)])


_sc_gather = functools.partial(
    pl.kernel,
    out_type=jax.ShapeDtypeStruct((3 * _N,), jnp.float32),
    mesh=plsc.VectorSubcoreMesh(
        core_axis_name="c", subcore_axis_name="s", num_cores=_NC, num_subcores=_NS
    ),
    scratch_types=[
        pltpu.VMEM((_NCHUNK, 1, _CH), jnp.int32),
        pltpu.VMEM((3, _NCHUNK, _CH), jnp.int32),
        pltpu.VMEM((3, _BPW), jnp.float32),
        pltpu.SemaphoreType.DMA,
    ],
    compiler_params=pltpu.CompilerParams(use_tc_tiling_on_sc=False),
)(_sc_gather_body)


_SBLK = 4096
_RW = _SBLK // 1024


def _tc_pad_body(scale_ref, x_ref, y_ref, z_ref, out_ref):
    s = scale_ref[0, 0]
    out_ref[...] = jnp.zeros_like(out_ref)
    for ch, ref in ((0, x_ref), (1, y_ref), (2, z_ref)):
        for r in range(_RW):
            pt = ref[r].T * s
            for j in range(8):
                out_ref[pl.ds(1024 * r + 128 * j, 128), ch : ch + 1] = pt[
                    :, j : j + 1
                ]


def kernel(token_ids, spiral_coords, radial_scale):
    ids3 = token_ids.reshape(_B, _S // 128, 128).transpose(1, 0, 2)
    table1 = spiral_coords.T.reshape(_VOCAB * 3)
    coords = _sc_gather(ids3, table1)
    coords8 = coords.reshape(3 * _N // 1024, 8, 128)
    scale = radial_scale.reshape(1, 1)

    nblk = _N // _SBLK
    plane = _N // 1024

    out = pl.pallas_call(
        _tc_pad_body,
        grid=(nblk,),
        in_specs=[
            pl.BlockSpec((1, 1), lambda i: (0, 0), memory_space=pltpu.SMEM),
            pl.BlockSpec((_RW, 8, 128), lambda i: (i, 0, 0)),
            pl.BlockSpec((_RW, 8, 128), lambda i: (i + plane // _RW, 0, 0)),
            pl.BlockSpec((_RW, 8, 128), lambda i: (i + 2 * plane // _RW, 0, 0)),
        ],
        out_specs=pl.BlockSpec((_SBLK, _D_MODEL), lambda i: (i, 0)),
        out_shape=jax.ShapeDtypeStruct((_N, _D_MODEL), jnp.float32),
    )(scale, coords8, coords8, coords8)

    return out.reshape(_B, _S, _D_MODEL)


# speedup vs baseline: 1.0366x; 1.0366x over previous
import functools

import jax
import jax.numpy as jnp
from jax import lax
from jax.experimental import pallas as pl
from jax.experimental.pallas import tpu as pltpu
from jax.experimental.pallas import tpu_sc as plsc

_VOCAB = 50257
_D_MODEL = 768
_B = 4
_S = 8192
_N = _B * _S

_NC = 2
_NS = 16
_NW = _NC * _NS
_BPW = _N // _NW
_CH = 128
_NCHUNK = _BPW // _CH


def _sc_gather_body(ids_hbm, table_hbm, out_hbm, idx_v, offs_v, g_v, sem):
    wid = lax.axis_index("s") * _NC + lax.axis_index("c")
    base = wid * _BPW
    bidx = wid // 8
    ccol0 = 8 * (wid % 8)
    pltpu.sync_copy(ids_hbm.at[pl.ds(ccol0, 8), pl.ds(bidx, 1), :], idx_v)

    def _offs_chunk(j, carry):
        for k in range(8):
            sl = pl.ds(16 * k, 16)
            t = idx_v[j, 0, sl]
            for ch in range(3):
                offs_v[ch, j, sl] = t + ch * _VOCAB
        return carry

    lax.fori_loop(0, _NCHUNK, _offs_chunk, 0)

    def _gather_chunk(j, carry):
        for ch in range(3):
            pltpu.async_copy(
                table_hbm.at[offs_v.at[ch, j]],
                g_v.at[ch, pl.ds(_CH * j, _CH)],
                sem,
            )
        return carry

    lax.fori_loop(0, _NCHUNK, _gather_chunk, 0)
    for ch in range(3):
        pltpu.make_async_copy(table_hbm.at[pl.ds(0, _BPW)], g_v.at[ch], sem).wait()
    for ch in range(3):
        pltpu.sync_copy(g_v.at[ch], out_hbm.at[pl.ds(ch * _N + base, _BPW)])


_sc_gather = functools.partial(
    pl.kernel,
    out_type=jax.ShapeDtypeStruct((3 * _N,), jnp.float32),
    mesh=plsc.VectorSubcoreMesh(
        core_axis_name="c", subcore_axis_name="s", num_cores=_NC, num_subcores=_NS
    ),
    scratch_types=[
        pltpu.VMEM((_NCHUNK, 1, _CH), jnp.int32),
        pltpu.VMEM((3, _NCHUNK, _CH), jnp.int32),
        pltpu.VMEM((3, _BPW), jnp.float32),
        pltpu.SemaphoreType.DMA,
    ],
    compiler_params=pltpu.CompilerParams(use_tc_tiling_on_sc=False),
)(_sc_gather_body)


_SBLK = 2048
_RW = _SBLK // 1024


def _tc_pad_body(scale_ref, x_ref, y_ref, z_ref, out_ref):
    s = scale_ref[0, 0]
    out_ref[...] = jnp.zeros_like(out_ref)
    for ch, ref in ((0, x_ref), (1, y_ref), (2, z_ref)):
        for r in range(_RW):
            pt = ref[r].T * s
            for j in range(8):
                out_ref[pl.ds(1024 * r + 128 * j, 128), ch : ch + 1] = pt[
                    :, j : j + 1
                ]


def kernel(token_ids, spiral_coords, radial_scale):
    ids3 = token_ids.reshape(_B, _S // 128, 128).transpose(1, 0, 2)
    table1 = spiral_coords.T.reshape(_VOCAB * 3)
    coords = _sc_gather(ids3, table1)
    coords8 = coords.reshape(3 * _N // 1024, 8, 128)
    scale = radial_scale.reshape(1, 1)

    nblk = _N // _SBLK
    plane = _N // 1024

    out = pl.pallas_call(
        _tc_pad_body,
        grid=(nblk,),
        in_specs=[
            pl.BlockSpec((1, 1), lambda i: (0, 0), memory_space=pltpu.SMEM),
            pl.BlockSpec((_RW, 8, 128), lambda i: (i, 0, 0)),
            pl.BlockSpec((_RW, 8, 128), lambda i: (i + plane // _RW, 0, 0)),
            pl.BlockSpec((_RW, 8, 128), lambda i: (i + 2 * plane // _RW, 0, 0)),
        ],
        out_specs=pl.BlockSpec((_SBLK, _D_MODEL), lambda i: (i, 0)),
        out_shape=jax.ShapeDtypeStruct((_N, _D_MODEL), jnp.float32),
    )(scale, coords8, coords8, coords8)

    return out.reshape(_B, _S, _D_MODEL)
